# Initial kernel scaffold; baseline (speedup 1.0000x reference)
import jax
import jax.numpy as jnp
from jax.experimental import pallas as pl


def kernel(x, edge_index, edge_attr, batch, Wn2l, bn2l, We2l, be2l, c1_Wn, c1_bn, c1_We, c1_be, c1_Wu, c1_bu, c2_Wn, c2_bn, c2_We, c2_be, c2_Wu, c2_bu, m1_W, m1_b, m2_W, m2_b):
    raise NotImplementedError("write your pallas kernel here")


# speedup vs baseline: 5.7140x; 5.7140x over previous
import functools

import jax
import jax.numpy as jnp
from jax import lax
from jax.experimental import pallas as pl
from jax.experimental.pallas import tpu as pltpu
from jax.experimental.pallas import tpu_sc as plsc

_NC = 2
_NS = 16
_CH = 80
_ZR = 64

_F32 = jnp.float32
_HIGH = jax.lax.Precision.HIGHEST


def _dot(a, b):
    return jax.lax.dot_general(a, b, (((1,), (0,)), ((), ())),
                               precision=_HIGH, preferred_element_type=_F32)


def _make_edge_pass(n_acc, d, e_pad):
    per_tile = e_pad // (_NC * _NS)
    n_chunks = per_tile // _CH
    rows_per_tile = n_acc // _NS
    nz = rows_per_tile // _ZR
    mesh = plsc.VectorSubcoreMesh(core_axis_name="c", subcore_axis_name="s")

    @functools.partial(
        pl.kernel,
        out_type=(jax.ShapeDtypeStruct((_NC, n_acc, d), _F32),
                  jax.ShapeDtypeStruct((_NC, n_acc, d), _F32)),
        mesh=mesh,
        scratch_types=[
            pltpu.VMEM((_CH,), jnp.int32),
            pltpu.VMEM((_CH, d), _F32),
            pltpu.VMEM((_CH, d), _F32),
            pltpu.VMEM((_ZR, d), _F32),
            pltpu.VMEM_SHARED((n_acc, d), _F32),
            pltpu.VMEM_SHARED((n_acc, d), _F32),
        ],
    )
    def edge_pass(ea_hbm, dst_hbm, ea_out, cnt_out,
                  dst_v, rows_v, ones_v, zbuf, acc_e, acc_c):
        c = lax.axis_index("c")
        s = lax.axis_index("s")

        @pl.loop(0, _ZR)
        def _(r):
            @pl.loop(0, d // 16)
            def _(k):
                zbuf[r, pl.ds(k * 16, 16)] = jnp.zeros((16,), _F32)

        @pl.loop(0, _CH)
        def _(r):
            @pl.loop(0, d // 16)
            def _(k):
                ones_v[r, pl.ds(k * 16, 16)] = jnp.ones((16,), _F32)

        base_z = s * rows_per_tile

        @pl.loop(0, nz)
        def _(z):
            pltpu.sync_copy(zbuf, acc_e.at[pl.ds(base_z + z * _ZR, _ZR)])
            pltpu.sync_copy(zbuf, acc_c.at[pl.ds(base_z + z * _ZR, _ZR)])

        plsc.subcore_barrier()

        base_e = (c * _NS + s) * per_tile

        @pl.loop(0, n_chunks)
        def _(i):
            off = base_e + i * _CH
            pltpu.sync_copy(dst_hbm.at[pl.ds(off, _CH)], dst_v)
            pltpu.sync_copy(ea_hbm.at[pl.ds(off, _CH)], rows_v)
            pltpu.sync_copy(rows_v, acc_e.at[dst_v], add=True)
            pltpu.sync_copy(ones_v, acc_c.at[dst_v], add=True)

        plsc.subcore_barrier()
        pltpu.sync_copy(acc_e.at[pl.ds(base_z, rows_per_tile)],
                        ea_out.at[c, pl.ds(base_z, rows_per_tile)])
        pltpu.sync_copy(acc_c.at[pl.ds(base_z, rows_per_tile)],
                        cnt_out.at[c, pl.ds(base_z, rows_per_tile)])

    return edge_pass


def _make_gather_pass(n_acc, d, e_pad):
    per_tile = e_pad // (_NC * _NS)
    n_chunks = per_tile // _CH
    rows_per_tile = n_acc // _NS
    nz = rows_per_tile // _ZR
    mesh = plsc.VectorSubcoreMesh(core_axis_name="c", subcore_axis_name="s")

    @functools.partial(
        pl.kernel,
        out_type=jax.ShapeDtypeStruct((_NC, n_acc, d), _F32),
        mesh=mesh,
        scratch_types=[
            pltpu.VMEM((_CH,), jnp.int32),
            pltpu.VMEM((_CH,), jnp.int32),
            pltpu.VMEM((_CH, d), _F32),
            pltpu.VMEM((_ZR, d), _F32),
            pltpu.VMEM_SHARED((n_acc, d), _F32),
            pltpu.SemaphoreType.DMA,
        ],
    )
    def gather_pass(h_hbm, src_hbm, dst_hbm, out_hbm,
                    src_v, dst_v, rows_v, zbuf, acc, sem):
        c = lax.axis_index("c")
        s = lax.axis_index("s")

        @pl.loop(0, _ZR)
        def _(r):
            @pl.loop(0, d // 16)
            def _(k):
                zbuf[r, pl.ds(k * 16, 16)] = jnp.zeros((16,), _F32)

        base_z = s * rows_per_tile

        @pl.loop(0, nz)
        def _(z):
            pltpu.sync_copy(zbuf, acc.at[pl.ds(base_z + z * _ZR, _ZR)])

        plsc.subcore_barrier()

        base_e = (c * _NS + s) * per_tile

        @pl.loop(0, n_chunks)
        def _(i):
            off = base_e + i * _CH
            pltpu.sync_copy(src_hbm.at[pl.ds(off, _CH)], src_v)
            pltpu.sync_copy(dst_hbm.at[pl.ds(off, _CH)], dst_v)
            pltpu.async_copy(h_hbm.at[src_v], rows_v, sem).wait()
            pltpu.sync_copy(rows_v, acc.at[dst_v], add=True)

        plsc.subcore_barrier()
        pltpu.sync_copy(acc.at[pl.ds(base_z, rows_per_tile)],
                        out_hbm.at[c, pl.ds(base_z, rows_per_tile)])

    return gather_pass


def _make_precompute(n, n_acc, h, de, blk):
    grid = (n // blk,)

    def body(x_ref, ea0, ea1, cn0, cn1, wn_ref, bn_ref,
             g1_ref, p1_ref, q1_ref, g2_ref, p2_ref, q2_ref,
             h0_ref, b1_ref, b2_ref):
        xb = x_ref[...]
        h0_ref[...] = _dot(xb, wn_ref[...]) + bn_ref[...]
        ea = ea0[0] + ea1[0]
        cnt = cn0[0][:, :1] + cn1[0][:, :1]
        b1_ref[...] = cnt * p1_ref[...] + _dot(ea, g1_ref[...]) + q1_ref[...]
        b2_ref[...] = cnt * p2_ref[...] + _dot(ea, g2_ref[...]) + q2_ref[...]

    full = lambda shape: pl.BlockSpec(shape, lambda i: tuple(0 for _ in shape))
    return pl.pallas_call(
        body,
        grid=grid,
        in_specs=[
            pl.BlockSpec((blk, h), lambda i: (i, 0)),
            pl.BlockSpec((1, blk, de), lambda i: (0, i, 0)),
            pl.BlockSpec((1, blk, de), lambda i: (1, i, 0)),
            pl.BlockSpec((1, blk, de), lambda i: (0, i, 0)),
            pl.BlockSpec((1, blk, de), lambda i: (1, i, 0)),
            full((h, h)), full((1, h)),
            full((de, h)), full((1, h)), full((1, h)),
            full((de, h)), full((1, h)), full((1, h)),
        ],
        out_specs=[
            pl.BlockSpec((blk, h), lambda i: (i, 0)),
            pl.BlockSpec((blk, h), lambda i: (i, 0)),
            pl.BlockSpec((blk, h), lambda i: (i, 0)),
        ],
        out_shape=[jax.ShapeDtypeStruct((n, h), _F32)] * 3,
    )


def _make_conv(n, n_acc, h, blk):
    grid = (n // blk,)

    def body(s0_ref, s1_ref, hp_ref, w_ref, b_ref, out_ref):
        agg = s0_ref[0] + s1_ref[0] + hp_ref[...]
        out_ref[...] = jnp.maximum(_dot(agg, w_ref[...]) + b_ref[...], 0.0)

    return pl.pallas_call(
        body,
        grid=grid,
        in_specs=[
            pl.BlockSpec((1, blk, h), lambda i: (0, i, 0)),
            pl.BlockSpec((1, blk, h), lambda i: (1, i, 0)),
            pl.BlockSpec((blk, h), lambda i: (i, 0)),
            pl.BlockSpec((h, h), lambda i: (0, 0)),
            pl.BlockSpec((blk, h), lambda i: (i, 0)),
        ],
        out_specs=pl.BlockSpec((blk, h), lambda i: (i, 0)),
        out_shape=jax.ShapeDtypeStruct((n, h), _F32),
    )


def _make_pool(n, h, num_graphs, out_dim, blk):
    grid = (n // blk,)
    last = n // blk - 1

    def body(h_ref, b_ref, m1_ref, m1b_ref, m2_ref, m2b_ref, out_ref,
             sums, counts, maxs):
        i = pl.program_id(0)

        @pl.when(i == 0)
        def _():
            sums[...] = jnp.zeros((num_graphs, h), _F32)
            counts[...] = jnp.zeros((num_graphs, h), _F32)
            maxs[...] = jnp.full((num_graphs, h), -jnp.inf, _F32)

        hb = h_ref[...]
        bb = b_ref[...]
        iota = lax.broadcasted_iota(jnp.int32, (1, num_graphs), 1)
        onehot = (bb == iota).astype(_F32)
        sums[...] += jax.lax.dot_general(
            onehot, hb, (((0,), (0,)), ((), ())),
            precision=_HIGH, preferred_element_type=_F32)
        counts[...] += jax.lax.dot_general(
            onehot, jnp.ones((blk, h), _F32), (((0,), (0,)), ((), ())),
            precision=_HIGH, preferred_element_type=_F32)
        for g in range(num_graphs):
            masked = jnp.where(bb == g, hb, -jnp.inf)
            m = jnp.max(masked, axis=0, keepdims=True)
            maxs[g:g + 1, :] = jnp.maximum(maxs[g:g + 1, :], m)

        @pl.when(i == last)
        def _():
            mean = sums[...] / jnp.maximum(counts[...], 1.0)
            z = (_dot(mean, m1_ref[:h, :]) + _dot(maxs[...], m1_ref[h:, :])
                 + m1b_ref[...])
            z = jnp.maximum(z, 0.0)
            o = _dot(z, m2_ref[...]) + m2b_ref[...]
            out_ref[...] = jax.nn.sigmoid(o)

    return pl.pallas_call(
        body,
        grid=grid,
        in_specs=[
            pl.BlockSpec((blk, h), lambda i: (i, 0)),
            pl.BlockSpec((blk, 1), lambda i: (i, 0)),
            pl.BlockSpec((2 * h, h), lambda i: (0, 0)),
            pl.BlockSpec((1, h), lambda i: (0, 0)),
            pl.BlockSpec((h, out_dim), lambda i: (0, 0)),
            pl.BlockSpec((1, out_dim), lambda i: (0, 0)),
        ],
        out_specs=pl.BlockSpec((num_graphs, out_dim), lambda i: (0, 0)),
        out_shape=jax.ShapeDtypeStruct((num_graphs, out_dim), _F32),
        scratch_shapes=[
            pltpu.VMEM((num_graphs, h), _F32),
            pltpu.VMEM((num_graphs, h), _F32),
            pltpu.VMEM((num_graphs, h), _F32),
        ],
    )


def kernel(x, edge_index, edge_attr, batch,
           Wn2l, bn2l, We2l, be2l,
           c1_Wn, c1_bn, c1_We, c1_be, c1_Wu, c1_bu,
           c2_Wn, c2_bn, c2_We, c2_be, c2_Wu, c2_bu,
           m1_W, m1_b, m2_W, m2_b):
    n, d_node = x.shape
    e = edge_index.shape[1]
    d_edge = edge_attr.shape[1]
    h = Wn2l.shape[1]
    num_graphs = 64
    out_dim = m2_W.shape[1]

    align = _NC * _NS * _CH
    e_pad = ((e + align - 1) // align) * align
    src = edge_index[0]
    dst = edge_index[1]
    ea = edge_attr
    if e_pad != e:
        pad = e_pad - e
        src = jnp.concatenate([src, jnp.zeros((pad,), jnp.int32)])
        dst = jnp.concatenate([dst, jnp.full((pad,), n, jnp.int32)])
        ea = jnp.concatenate([ea, jnp.zeros((pad, d_edge), _F32)], axis=0)

    n_acc = ((n + 1 + 16 * _ZR - 1) // (16 * _ZR)) * (16 * _ZR)

    W1 = c1_Wn @ c1_Wu
    W2 = c2_Wn @ c2_Wu
    W1e = c1_We @ c1_Wu
    W2e = c2_We @ c2_Wu
    u1 = (c1_bn + c1_be) @ c1_Wu
    u2 = (c2_bn + c2_be) @ c2_Wu
    G1 = We2l @ W1e
    G2 = We2l @ W2e
    p1 = (u1 + be2l @ W1e).reshape(1, h)
    p2 = (u2 + be2l @ W2e).reshape(1, h)
    q1 = (u1 + c1_bu).reshape(1, h)
    q2 = (u2 + c2_bu).reshape(1, h)

    edge_pass = _make_edge_pass(n_acc, d_edge, e_pad)
    gather_pass = _make_gather_pass(n_acc, h, e_pad)
    blk = 1000
    precompute = _make_precompute(n, n_acc, h, d_edge, blk)
    conv = _make_conv(n, n_acc, h, blk)
    pool = _make_pool(n, h, num_graphs, out_dim, blk)

    ea_part, cnt_part = edge_pass(ea, dst)
    h0, B1, B2 = precompute(x, ea_part, ea_part, cnt_part, cnt_part,
                            Wn2l, bn2l.reshape(1, h),
                            G1, p1, q1, G2, p2, q2)

    hcur = h0
    for W, B in ((W1, B1), (W2, B2)) * 3:
        s_part = gather_pass(hcur, src, dst)
        hcur = conv(s_part, s_part, hcur, W, B)

    return pool(hcur, batch.reshape(n, 1).astype(jnp.int32),
                m1_W, m1_b.reshape(1, h),
                m2_W, m2_b.reshape(1, out_dim))
